# Initial kernel scaffold; baseline (speedup 1.0000x reference)
import functools

import jax
import jax.numpy as jnp
from jax import lax
from jax.experimental import pallas as pl
from jax.experimental.pallas import tpu as pltpu
from jax.experimental.pallas import tpu_sc as plsc

N_DATA = 50000
N_TRUNC = 12500
E = 800000
F = 44
W = 32
NC = 2
NS = 16
LANES = 16

EPR = 128
WR = 4
WIN = EPR * WR
ROWS = 6272
E_PAD = ROWS * EPR
ROWS_PT = ROWS // NS
NWIN = ROWS_PT // WR

ZROWS = 512


def _zero_span(s, nrows):
    q, r = divmod(nrows, NS)
    lo = s * q + jnp.minimum(s, r)
    sz = q + (s < r).astype(jnp.int32)
    return lo, sz


def _body(xpad, dsrc, ddst, dwt, usrc, udst, uwt, xtr, out,
          acc_dn, acc_up, idx_v, dst_v, w_v, rows_v, zbuf, sem):
    c = lax.axis_index("c")
    s = lax.axis_index("s")

    zvec = jnp.zeros((LANES,), jnp.float32)

    @pl.loop(0, ZROWS)
    def _(rr):
        zbuf[rr, pl.ds(0, LANES)] = zvec
        zbuf[rr, pl.ds(LANES, LANES)] = zvec

    def zero_acc(acc, nrows, nz):
        lo, sz = _zero_span(s, nrows)
        hi = lo + sz
        for i in range(nz):
            st = jnp.minimum(lo + i * ZROWS, hi - ZROWS)
            pltpu.sync_copy(zbuf, acc.at[pl.ds(st, ZROWS)])

    def copy_out(acc, nrows, nz, dst_hbm, dst_base):
        lo, sz = _zero_span(s, nrows)
        hi = lo + sz
        for i in range(nz):
            st = jnp.minimum(lo + i * ZROWS, hi - ZROWS)
            pltpu.sync_copy(acc.at[pl.ds(st, ZROWS)],
                            dst_hbm.at[pl.ds(dst_base + st, ZROWS)])

    def stage(src_r, dst_r, wt_r, table, tbl_base, acc, acc_rows, nz,
              out_hbm, out_base):
        zero_acc(acc, acc_rows, nz)
        plsc.subcore_barrier()

        rowbase = s * ROWS_PT
        offv = jnp.broadcast_to(tbl_base, (LANES,)).astype(jnp.int32)

        @pl.loop(0, NWIN)
        def _(wi):
            wrow = rowbase + wi * WR
            pltpu.sync_copy(src_r.at[pl.ds(wrow, WR)], idx_v)
            pltpu.sync_copy(dst_r.at[pl.ds(wrow, WR)], dst_v)
            pltpu.sync_copy(wt_r.at[pl.ds(wrow, WR)], w_v)
            for r in range(WR):
                for k in range(EPR // LANES):
                    sl = pl.ds(k * LANES, LANES)
                    idx_v[r, sl] = idx_v[r, sl] + offv
            cps = [
                pltpu.async_copy(table.at[idx_v.at[r]],
                                 rows_v.at[pl.ds(r * EPR, EPR)], sem)
                for r in range(WR)
            ]
            for cp in cps:
                cp.wait()

            @pl.loop(0, WR)
            def _(r):
                for k in range(EPR // LANES):
                    wv = w_v[r, pl.ds(k * LANES, LANES)]
                    for i in range(LANES):
                        e = r * EPR + k * LANES + i
                        ws = jnp.broadcast_to(wv[i], (LANES,))
                        rows_v[e, pl.ds(0, LANES)] = (
                            rows_v[e, pl.ds(0, LANES)] * ws)
                        rows_v[e, pl.ds(LANES, LANES)] = (
                            rows_v[e, pl.ds(LANES, LANES)] * ws)

            for r in range(WR):
                pltpu.sync_copy(rows_v.at[pl.ds(r * EPR, EPR)],
                                acc.at[dst_v.at[r]], add=True)

        plsc.subcore_barrier()
        copy_out(acc, acc_rows, nz, out_hbm, out_base)
        plsc.subcore_barrier()

    for b in range(2):
        bh = b * NC + c
        stage(dsrc, ddst, dwt, xpad, bh * N_DATA, acc_dn, N_TRUNC, 2,
              xtr, bh * N_TRUNC)
        stage(usrc, udst, uwt, xtr, bh * N_TRUNC, acc_up, N_DATA, 7,
              out, bh * N_DATA)


@jax.jit
def _run(xpad, dsrc, ddst, dwt, usrc, udst, uwt):
    mesh = plsc.VectorSubcoreMesh(core_axis_name="c", subcore_axis_name="s")
    f = pl.kernel(
        _body,
        out_type=(
            jax.ShapeDtypeStruct((2 * NC * N_TRUNC, W), jnp.float32),
            jax.ShapeDtypeStruct((2 * NC * N_DATA, W), jnp.float32),
        ),
        mesh=mesh,
        scratch_types=[
            pltpu.VMEM_SHARED((N_TRUNC, W), jnp.float32),
            pltpu.VMEM_SHARED((N_DATA, W), jnp.float32),
            pltpu.VMEM((WR, EPR), jnp.int32),
            pltpu.VMEM((WR, EPR), jnp.int32),
            pltpu.VMEM((WR, EPR), jnp.float32),
            pltpu.VMEM((WIN, W), jnp.float32),
            pltpu.VMEM((ZROWS, W), jnp.float32),
            pltpu.SemaphoreType.DMA,
        ],
    )
    return f(xpad, dsrc, ddst, dwt, usrc, udst, uwt)


def _pad_edges(a):
    return jnp.concatenate(
        [a, jnp.zeros((E_PAD - E,), a.dtype)]).reshape(ROWS, EPR)


def kernel(x, down_src, down_dst, down_weight, up_src, up_dst, up_weight):
    b, t, en, n, f = x.shape
    x2 = x.reshape(b * t * en, n, f)
    xpad = jnp.pad(x2, ((0, 0), (0, 0), (0, 2 * W - f)))
    xpad = xpad.reshape(b * t * en, n, NC, W).transpose(0, 2, 1, 3)
    xpad = xpad.reshape(b * t * en * NC * n, W)

    _, outp = _run(
        xpad,
        _pad_edges(down_src), _pad_edges(down_dst),
        _pad_edges(down_weight),
        _pad_edges(up_src), _pad_edges(up_dst), _pad_edges(up_weight),
    )
    outp = outp.reshape(b * t * en, NC, n, W).transpose(0, 2, 1, 3)
    outp = outp.reshape(b * t * en, n, NC * W)[:, :, :f]
    return outp.reshape(b, t, en, n, f)


# speedup vs baseline: 75.8377x; 75.8377x over previous
import jax
import jax.numpy as jnp
from jax import lax
from jax.experimental import pallas as pl
from jax.experimental.pallas import tpu as pltpu
from jax.experimental.pallas import tpu_sc as plsc

N_DATA = 50000
N_TRUNC = 12500
ND_PAD = 51200
NT_PAD = 12800
AH_UP = ND_PAD // 2
E = 800000
F = 44
W = 32
NC = 2
NS = 16
LANES = 16

EPR = 128
WR = 8
WIN = EPR * WR
ROWS = 6272
E_PAD = ROWS * EPR
ROWS_PT = ROWS // NS
NWIN = ROWS_PT // WR

ZROWS = 256


def _body(xpad, dsrc, ddst, dwt, usrc, udst, uwt, xtr, out,
          acc_dn, acc_up, idx_v, dst_v, w_v, rows_v, zbuf, sem):
    c = lax.axis_index("c")
    s = lax.axis_index("s")

    zvec = jnp.zeros((LANES,), jnp.float32)

    @pl.loop(0, ZROWS)
    def _(rr):
        zbuf[rr, pl.ds(0, LANES)] = zvec
        zbuf[rr, pl.ds(LANES, LANES)] = zvec

    def spans(nrows, nz):
        q = nrows // NS
        lo = s * q
        res = []
        for i in range(nz):
            res.append(jnp.minimum(lo + i * ZROWS, lo + q - ZROWS))
        return res

    def zero_acc(acc, nrows, nz):
        for st in spans(nrows, nz):
            pltpu.sync_copy(zbuf, acc.at[pl.ds(st, ZROWS)])

    def copy_out(acc, nrows, nz, dst_hbm, dst_base):
        for st in spans(nrows, nz):
            pltpu.sync_copy(acc.at[pl.ds(st, ZROWS)],
                            dst_hbm.at[pl.ds(dst_base + st, ZROWS)])

    def stage(src_r, dst_r, wt_r, table, tbl_base, acc, acc_rows, nz,
              out_hbm, out_base, dlo, dhi):
        zero_acc(acc, acc_rows, nz)
        plsc.subcore_barrier()

        rowbase = s * ROWS_PT
        offv = jnp.broadcast_to(tbl_base, (LANES,)).astype(jnp.int32)
        dlov = jnp.broadcast_to(dlo, (LANES,)).astype(jnp.int32)
        dhiv = jnp.broadcast_to(dhi, (LANES,)).astype(jnp.int32)

        @pl.loop(0, NWIN)
        def _(wi):
            wrow = rowbase + wi * WR
            pltpu.sync_copy(src_r.at[pl.ds(wrow, WR)], idx_v)
            pltpu.sync_copy(dst_r.at[pl.ds(wrow, WR)], dst_v)
            pltpu.sync_copy(wt_r.at[pl.ds(wrow, WR)], w_v)

            d_first = dst_v[0, pl.ds(0, LANES)][0]
            d_last = dst_v[WR - 1, pl.ds(EPR - LANES, LANES)][LANES - 1]
            active = jnp.logical_and(d_first < dhi, d_last >= dlo)

            @pl.when(active)
            def _():
                @pl.loop(0, WR)
                def _(r):
                    for k in range(EPR // LANES):
                        sl = pl.ds(k * LANES, LANES)
                        idx_v[r, sl] = idx_v[r, sl] + offv
                        d = dst_v[r, sl]
                        inr = jnp.logical_and(d >= dlov, d < dhiv)
                        w_v[r, sl] = jnp.where(inr, w_v[r, sl], 0.0)
                        dst_v[r, sl] = jnp.clip(d - dlov, 0, acc_rows - 1)

                cps = [
                    pltpu.async_copy(table.at[idx_v.at[r]],
                                     rows_v.at[pl.ds(r * EPR, EPR)], sem)
                    for r in range(WR)
                ]
                for cp in cps:
                    cp.wait()

                @pl.loop(0, WR)
                def _(r):
                    for k in range(EPR // LANES):
                        wv = w_v[r, pl.ds(k * LANES, LANES)]
                        for i in range(LANES):
                            e = r * EPR + k * LANES + i
                            ws = jnp.broadcast_to(wv[i], (LANES,))
                            rows_v[e, pl.ds(0, LANES)] = (
                                rows_v[e, pl.ds(0, LANES)] * ws)
                            rows_v[e, pl.ds(LANES, LANES)] = (
                                rows_v[e, pl.ds(LANES, LANES)] * ws)

                for r in range(WR):
                    pltpu.sync_copy(rows_v.at[pl.ds(r * EPR, EPR)],
                                    acc.at[dst_v.at[r]], add=True)

        plsc.subcore_barrier()
        copy_out(acc, acc_rows, nz, out_hbm, out_base)
        plsc.subcore_barrier()

    @pl.loop(0, 2)
    def _(b):
        bh = b * NC + c
        stage(dsrc, ddst, dwt, xpad, bh * N_DATA, acc_dn, NT_PAD, 4,
              xtr, bh * NT_PAD, 0, NT_PAD)

        @pl.loop(0, 2)
        def _(h):
            dlo = h * AH_UP
            stage(usrc, udst, uwt, xtr, bh * NT_PAD, acc_up, AH_UP, 7,
                  out, bh * ND_PAD + dlo, dlo, dlo + AH_UP)


@jax.jit
def _run(xpad, dsrc, ddst, dwt, usrc, udst, uwt):
    mesh = plsc.VectorSubcoreMesh(core_axis_name="c", subcore_axis_name="s")
    f = pl.kernel(
        _body,
        out_type=(
            jax.ShapeDtypeStruct((2 * NC * NT_PAD, W), jnp.float32),
            jax.ShapeDtypeStruct((2 * NC * ND_PAD, W), jnp.float32),
        ),
        mesh=mesh,
        compiler_params=pltpu.CompilerParams(use_tc_tiling_on_sc=False),
        scratch_types=[
            pltpu.VMEM_SHARED((NT_PAD, W), jnp.float32),
            pltpu.VMEM_SHARED((AH_UP, W), jnp.float32),
            pltpu.VMEM((WR, EPR), jnp.int32),
            pltpu.VMEM((WR, EPR), jnp.int32),
            pltpu.VMEM((WR, EPR), jnp.float32),
            pltpu.VMEM((WIN, W), jnp.float32),
            pltpu.VMEM((ZROWS, W), jnp.float32),
            pltpu.SemaphoreType.DMA,
        ],
    )
    return f(xpad, dsrc, ddst, dwt, usrc, udst, uwt)


def _pad_edges(a, fill=0):
    return jnp.concatenate(
        [a, jnp.full((E_PAD - E,), fill, a.dtype)]).reshape(ROWS, EPR)


def kernel(x, down_src, down_dst, down_weight, up_src, up_dst, up_weight):
    b, t, en, n, f = x.shape
    x2 = x.reshape(b * t * en, n, f)
    xpad = jnp.pad(x2, ((0, 0), (0, 0), (0, 2 * W - f)))
    xpad = xpad.reshape(b * t * en, n, NC, W).transpose(0, 2, 1, 3)
    xpad = xpad.reshape(b * t * en * NC * n, W)

    _, outp = _run(
        xpad,
        _pad_edges(down_src), _pad_edges(down_dst, N_TRUNC - 1),
        _pad_edges(down_weight),
        _pad_edges(up_src), _pad_edges(up_dst, N_DATA - 1),
        _pad_edges(up_weight),
    )
    outp = outp.reshape(b * t * en, NC, ND_PAD, W)[:, :, :n]
    outp = outp.transpose(0, 2, 1, 3).reshape(b * t * en, n, NC * W)[:, :, :f]
    return outp.reshape(b, t, en, n, f)
